# baseline (speedup 1.0000x reference)
import functools

import jax
import jax.numpy as jnp
from jax import lax
from jax.experimental import pallas as pl
from jax.experimental.pallas import tpu as pltpu
from jax.experimental.pallas import tpu_sc as plsc

_NUM_USERS = 25000
_NUM_LAYERS = 4
_DIM = 64
_Q = 16
_N_NODES = 50000
_N_EDGES = 800000

_P = 50176
_ROWS_W = _P // 16
_NB = 392
_EPAD = 16 * _NB * 128
_KB = 8
_NBLK = _NB // _KB
_KP = 7
_NBLK2 = _NB // _KP // 2
_DUMP = 50000
_ZROWS = _ROWS_W // 4

_mesh = plsc.VectorSubcoreMesh(core_axis_name="c", subcore_axis_name="s")


@functools.partial(
    pl.kernel,
    out_type=jax.ShapeDtypeStruct((16, _P), jnp.float32),
    mesh=_mesh,
    scratch_types=[
        pltpu.VMEM((_P,), jnp.float32),
        pltpu.VMEM((_KB, 128), jnp.int32),
    ],
    compiler_params=pltpu.CompilerParams(
        needs_layout_passes=False, use_tc_tiling_on_sc=False),
)
def _deg_kernel(dst_h, degp_h, deg_v, dstb_v):
    c = lax.axis_index("c")
    s = lax.axis_index("s")

    @pl.when(c == 0)
    def _():
        zero16 = jnp.zeros((16,), jnp.float32)

        def zbody(i, _):
            deg_v[pl.ds(i * 16, 16)] = zero16
            return 0

        lax.fori_loop(0, _P // 16, zbody, 0)
        ones16 = jnp.ones((16,), jnp.float32)

        def blk(b, _):
            pltpu.sync_copy(dst_h.at[s, pl.ds(b * _KB, _KB)], dstb_v)
            for j in range(_KB):
                for k in range(8):
                    idx = dstb_v[j, pl.ds(k * 16, 16)]
                    plsc.addupdate_scatter(deg_v, [idx], ones16)
            return 0

        lax.fori_loop(0, _NBLK, blk, 0)
        pltpu.sync_copy(deg_v, degp_h.at[s])


@functools.partial(
    pl.kernel,
    out_type=(
        jax.ShapeDtypeStruct((_NUM_LAYERS, 4 * _P, _Q), jnp.float32),
        jax.ShapeDtypeStruct((2, 4 * _P, _Q), jnp.float32),
    ),
    mesh=_mesh,
    scratch_types=[
        pltpu.VMEM_SHARED((_P, _Q), jnp.float32),
        pltpu.VMEM((2, _KP, 128), jnp.int32),
        pltpu.VMEM((2, _KP, 128), jnp.int32),
        pltpu.VMEM((2, _KP, 128, _Q), jnp.float32),
        pltpu.VMEM((_ZROWS, _Q), jnp.float32),
        pltpu.VMEM((_ROWS_W,), jnp.float32),
        pltpu.SemaphoreType.DMA,
        pltpu.SemaphoreType.DMA,
    ],
    compiler_params=pltpu.CompilerParams(
        needs_layout_passes=False, use_tc_tiling_on_sc=False),
)
def _mega_kernel(z0_h, src_h, dst_h, d2_h, acc_h, zt_h, acc_sp, src_v, dst_v,
                 rows_v, cbuf_v, d2_v, gsem, ssem):
    c = lax.axis_index("c")
    s = lax.axis_index("s")
    row0 = s * _ROWS_W
    pltpu.sync_copy(d2_h.at[s], d2_v)
    zero16 = jnp.zeros((16,), jnp.float32)

    def zb(i, _):
        cbuf_v[i, pl.ds(0, 16)] = zero16
        return 0

    lax.fori_loop(0, _ZROWS, zb, 0)

    for l in range(_NUM_LAYERS):
        gref = z0_h if l == 0 else zt_h.at[(l + 1) % 2]

        def qsweep(q01, _, l=l, gref=gref):
            quarter = 2 * c + q01
            qoff = quarter * _P
            for r in range(4):
                pltpu.sync_copy(
                    cbuf_v, acc_sp.at[pl.ds(row0 + r * _ZROWS, _ZROWS)])
            plsc.subcore_barrier()

            def stage_fire(b, p):
                pltpu.sync_copy(src_h.at[quarter, s, pl.ds(b * _KP, _KP)],
                                src_v.at[p])
                pltpu.sync_copy(dst_h.at[s, pl.ds(b * _KP, _KP)],
                                dst_v.at[p])
                for j in range(_KP):
                    pltpu.async_copy(gref.at[src_v.at[p, j]],
                                     rows_v.at[p, j], gsem)

            def wait_gathers(p):
                for j in range(_KP):
                    pltpu.make_async_copy(gref.at[src_v.at[p, j]],
                                          rows_v.at[p, j], gsem).wait()

            def fire_scatters(p):
                for j in range(_KP):
                    pltpu.async_copy(rows_v.at[p, j],
                                     acc_sp.at[dst_v.at[p, j]], ssem,
                                     add=True)

            def wait_scatters(p):
                for j in range(_KP):
                    pltpu.make_async_copy(rows_v.at[p, j],
                                          acc_sp.at[dst_v.at[p, j]],
                                          ssem).wait()

            stage_fire(0, 0)

            def blk2(i, _):
                bB = 2 * i + 1
                stage_fire(bB, 1)
                wait_gathers(0)
                fire_scatters(0)
                wait_gathers(1)
                wait_scatters(0)
                bA2 = lax.rem(2 * i + 2, 2 * _NBLK2)
                stage_fire(bA2, 0)
                fire_scatters(1)
                wait_scatters(1)
                return 0

            lax.fori_loop(0, _NBLK2, blk2, 0)
            wait_gathers(0)
            plsc.subcore_barrier()

            for r in range(4):
                coff = row0 + r * _ZROWS
                pltpu.sync_copy(acc_sp.at[pl.ds(coff, _ZROWS)], cbuf_v)
                pltpu.sync_copy(cbuf_v, acc_h.at[l, pl.ds(qoff + coff,
                                                          _ZROWS)])
                if l < _NUM_LAYERS - 1:
                    def sc16(g, _, r=r):
                        base = r * _ZROWS + g * 16
                        for rr in range(16):
                            bc = plsc.load_gather(
                                d2_v,
                                [jnp.full((16,), base + rr, jnp.int32)])
                            row = cbuf_v[g * 16 + rr, pl.ds(0, 16)]
                            cbuf_v[g * 16 + rr, pl.ds(0, 16)] = row * bc
                        return 0

                    lax.fori_loop(0, _ZROWS // 16, sc16, 0)
                    pltpu.sync_copy(
                        cbuf_v, zt_h.at[l % 2, pl.ds(qoff + coff, _ZROWS)])
            lax.fori_loop(0, _ZROWS, zb, 0)
            return 0

        lax.fori_loop(0, 2, qsweep, 0)


_RB = 1024
_G = _P // _RB


def _emb_quarter(emb, q):
    lo = jnp.where(q == 0, emb[:, 0 * _Q:1 * _Q], emb[:, 1 * _Q:2 * _Q])
    hi = jnp.where(q == 2, emb[:, 2 * _Q:3 * _Q], emb[:, 3 * _Q:4 * _Q])
    return jnp.where(q < 2, lo, hi)


def _tc1_body(degp, emb, z, dinv, dinv2):
    q = pl.program_id(1)
    deg = jnp.sum(degp[...], axis=0)
    d = jnp.where(deg > 0, lax.rsqrt(jnp.where(deg > 0, deg, 1.0)), 0.0)
    z[...] = (_emb_quarter(emb[...], q) * d[:, None])[None]
    dinv[...] = d[:, None]
    dinv2[...] = (d * d)[:, None]


_tc1 = pl.pallas_call(
    _tc1_body,
    grid=(_G, 4),
    in_specs=[
        pl.BlockSpec((16, _RB), lambda g, q: (0, g)),
        pl.BlockSpec((_RB, _DIM), lambda g, q: (g, 0)),
    ],
    out_specs=[
        pl.BlockSpec((1, _RB, _Q), lambda g, q: (q, g, 0)),
        pl.BlockSpec((_RB, 1), lambda g, q: (g, 0)),
        pl.BlockSpec((_RB, 1), lambda g, q: (g, 0)),
    ],
    out_shape=[
        jax.ShapeDtypeStruct((4, _P, _Q), jnp.float32),
        jax.ShapeDtypeStruct((_P, 1), jnp.float32),
        jax.ShapeDtypeStruct((_P, 1), jnp.float32),
    ],
)


def _tc3_body(a0, a1, a2, a3, emb, dinv, out):
    qs = [jnp.sum(a[...], axis=0) for a in (a0, a1, a2, a3)]
    wide = jnp.concatenate(qs, axis=-1)
    out[...] = (1.0 / ((_NUM_LAYERS + 1) ** 2)) * (emb[...] + dinv[...] * wide)


_tc3 = pl.pallas_call(
    _tc3_body,
    grid=(_G,),
    in_specs=[
        pl.BlockSpec((_NUM_LAYERS, _RB, _Q),
                     functools.partial(lambda q, g: (0, q * _G + g, 0), q))
        for q in range(4)
    ]
    + [
        pl.BlockSpec((_RB, _DIM), lambda g: (g, 0)),
        pl.BlockSpec((_RB, 1), lambda g: (g, 0)),
    ],
    out_specs=pl.BlockSpec((_RB, _DIM), lambda g: (g, 0)),
    out_shape=jax.ShapeDtypeStruct((_P, _DIM), jnp.float32),
)


def kernel(edge_index, user_weight, item_weight):
    src = edge_index[0].astype(jnp.int32)
    dst = edge_index[1].astype(jnp.int32)
    pad = _EPAD - _N_EDGES
    src_p = jnp.concatenate([src, jnp.zeros((pad,), jnp.int32)])
    dst_p = jnp.concatenate([dst, jnp.full((pad,), _DUMP, jnp.int32)])
    offs = jnp.arange(4, dtype=jnp.int32)[:, None] * _P
    src4 = (src_p[None, :] + offs).reshape(4, 16, _NB, 128)
    dst_r = dst_p.reshape(16, _NB, 128)
    emb = jnp.concatenate(
        [user_weight, item_weight,
         jnp.zeros((_P - _N_NODES, _DIM), jnp.float32)], axis=0)

    degp = _deg_kernel(dst_r)
    z0, dinv, d2 = _tc1(degp, emb)
    d2r = d2[:, 0].reshape(16, _ROWS_W)
    acc, _ = _mega_kernel(z0.reshape(4 * _P, _Q), src4, dst_r, d2r)
    out = _tc3(acc, acc, acc, acc, emb, dinv)
    return out[:_NUM_USERS], out[_NUM_USERS:_N_NODES]


# speedup vs baseline: 1.1295x; 1.1295x over previous
import functools

import jax
import jax.numpy as jnp
from jax import lax
from jax.experimental import pallas as pl
from jax.experimental.pallas import tpu as pltpu
from jax.experimental.pallas import tpu_sc as plsc

_NUM_USERS = 25000
_NUM_LAYERS = 4
_DIM = 64
_Q = 16
_N_NODES = 50000
_N_EDGES = 800000

_P = 50176
_ROWS_W = _P // 16
_NB = 392
_EPAD = 16 * _NB * 128
_KB = 8
_NBLK = _NB // _KB
_KP = 14
_NBLK2 = _NB // _KP // 2
_DUMP = 50000
_ZROWS = _ROWS_W // 4

_mesh = plsc.VectorSubcoreMesh(core_axis_name="c", subcore_axis_name="s")


@functools.partial(
    pl.kernel,
    out_type=jax.ShapeDtypeStruct((16, _P), jnp.float32),
    mesh=_mesh,
    scratch_types=[
        pltpu.VMEM((_P,), jnp.float32),
        pltpu.VMEM((_KB, 128), jnp.int32),
    ],
    compiler_params=pltpu.CompilerParams(
        needs_layout_passes=False, use_tc_tiling_on_sc=False),
)
def _deg_kernel(dst_h, degp_h, deg_v, dstb_v):
    c = lax.axis_index("c")
    s = lax.axis_index("s")

    @pl.when(c == 0)
    def _():
        zero16 = jnp.zeros((16,), jnp.float32)

        def zbody(i, _):
            deg_v[pl.ds(i * 16, 16)] = zero16
            return 0

        lax.fori_loop(0, _P // 16, zbody, 0)
        ones16 = jnp.ones((16,), jnp.float32)

        def blk(b, _):
            pltpu.sync_copy(dst_h.at[s, pl.ds(b * _KB, _KB)], dstb_v)
            for j in range(_KB):
                for k in range(8):
                    idx = dstb_v[j, pl.ds(k * 16, 16)]
                    plsc.addupdate_scatter(deg_v, [idx], ones16)
            return 0

        lax.fori_loop(0, _NBLK, blk, 0)
        pltpu.sync_copy(deg_v, degp_h.at[s])


@functools.partial(
    pl.kernel,
    out_type=(
        jax.ShapeDtypeStruct((_NUM_LAYERS, 4 * _P, _Q), jnp.float32),
        jax.ShapeDtypeStruct((2, 4 * _P, _Q), jnp.float32),
    ),
    mesh=_mesh,
    scratch_types=[
        pltpu.VMEM_SHARED((_P, _Q), jnp.float32),
        pltpu.VMEM((2, _KP, 128), jnp.int32),
        pltpu.VMEM((2, _KP, 128), jnp.int32),
        pltpu.VMEM((2, _KP, 128, _Q), jnp.float32),
        pltpu.VMEM((_ZROWS, _Q), jnp.float32),
        pltpu.VMEM((_ROWS_W,), jnp.float32),
        pltpu.SemaphoreType.DMA,
        pltpu.SemaphoreType.DMA,
    ],
    compiler_params=pltpu.CompilerParams(
        needs_layout_passes=False, use_tc_tiling_on_sc=False),
)
def _mega_kernel(z0_h, src_h, dst_h, d2_h, acc_h, zt_h, acc_sp, src_v, dst_v,
                 rows_v, cbuf_v, d2_v, gsem, ssem):
    c = lax.axis_index("c")
    s = lax.axis_index("s")
    row0 = s * _ROWS_W
    pltpu.sync_copy(d2_h.at[s], d2_v)
    zero16 = jnp.zeros((16,), jnp.float32)

    def zb(i, _):
        cbuf_v[i, pl.ds(0, 16)] = zero16
        return 0

    lax.fori_loop(0, _ZROWS, zb, 0)

    for l in range(_NUM_LAYERS):
        gref = z0_h if l == 0 else zt_h.at[(l + 1) % 2]

        def qsweep(q01, _, l=l, gref=gref):
            quarter = 2 * c + q01
            qoff = quarter * _P
            for r in range(4):
                pltpu.sync_copy(
                    cbuf_v, acc_sp.at[pl.ds(row0 + r * _ZROWS, _ZROWS)])
            plsc.subcore_barrier()

            def stage_fire(b, p):
                pltpu.sync_copy(src_h.at[quarter, s, pl.ds(b * _KP, _KP)],
                                src_v.at[p])
                pltpu.sync_copy(dst_h.at[s, pl.ds(b * _KP, _KP)],
                                dst_v.at[p])
                for j in range(_KP):
                    pltpu.async_copy(gref.at[src_v.at[p, j]],
                                     rows_v.at[p, j], gsem)

            def wait_gathers(p):
                for j in range(_KP):
                    pltpu.make_async_copy(gref.at[src_v.at[p, j]],
                                          rows_v.at[p, j], gsem).wait()

            def fire_scatters(p):
                for j in range(_KP):
                    pltpu.async_copy(rows_v.at[p, j],
                                     acc_sp.at[dst_v.at[p, j]], ssem,
                                     add=True)

            def wait_scatters(p):
                for j in range(_KP):
                    pltpu.make_async_copy(rows_v.at[p, j],
                                          acc_sp.at[dst_v.at[p, j]],
                                          ssem).wait()

            stage_fire(0, 0)

            def blk2(i, _):
                bB = 2 * i + 1
                stage_fire(bB, 1)
                wait_gathers(0)
                fire_scatters(0)
                wait_gathers(1)
                wait_scatters(0)
                bA2 = lax.rem(2 * i + 2, 2 * _NBLK2)
                stage_fire(bA2, 0)
                fire_scatters(1)
                wait_scatters(1)
                return 0

            lax.fori_loop(0, _NBLK2, blk2, 0)
            wait_gathers(0)
            plsc.subcore_barrier()

            for r in range(4):
                coff = row0 + r * _ZROWS
                pltpu.sync_copy(acc_sp.at[pl.ds(coff, _ZROWS)], cbuf_v)
                pltpu.sync_copy(cbuf_v, acc_h.at[l, pl.ds(qoff + coff,
                                                          _ZROWS)])
                if l < _NUM_LAYERS - 1:
                    def sc16(g, _, r=r):
                        base = r * _ZROWS + g * 16
                        for rr in range(16):
                            bc = plsc.load_gather(
                                d2_v,
                                [jnp.full((16,), base + rr, jnp.int32)])
                            row = cbuf_v[g * 16 + rr, pl.ds(0, 16)]
                            cbuf_v[g * 16 + rr, pl.ds(0, 16)] = row * bc
                        return 0

                    lax.fori_loop(0, _ZROWS // 16, sc16, 0)
                    pltpu.sync_copy(
                        cbuf_v, zt_h.at[l % 2, pl.ds(qoff + coff, _ZROWS)])
            lax.fori_loop(0, _ZROWS, zb, 0)
            return 0

        lax.fori_loop(0, 2, qsweep, 0)


_RB = 1024
_G = _P // _RB


def _emb_quarter(emb, q):
    lo = jnp.where(q == 0, emb[:, 0 * _Q:1 * _Q], emb[:, 1 * _Q:2 * _Q])
    hi = jnp.where(q == 2, emb[:, 2 * _Q:3 * _Q], emb[:, 3 * _Q:4 * _Q])
    return jnp.where(q < 2, lo, hi)


def _tc1_body(degp, emb, z, dinv, dinv2):
    q = pl.program_id(1)
    deg = jnp.sum(degp[...], axis=0)
    d = jnp.where(deg > 0, lax.rsqrt(jnp.where(deg > 0, deg, 1.0)), 0.0)
    z[...] = (_emb_quarter(emb[...], q) * d[:, None])[None]
    dinv[...] = d[:, None]
    dinv2[...] = (d * d)[:, None]


_tc1 = pl.pallas_call(
    _tc1_body,
    grid=(_G, 4),
    in_specs=[
        pl.BlockSpec((16, _RB), lambda g, q: (0, g)),
        pl.BlockSpec((_RB, _DIM), lambda g, q: (g, 0)),
    ],
    out_specs=[
        pl.BlockSpec((1, _RB, _Q), lambda g, q: (q, g, 0)),
        pl.BlockSpec((_RB, 1), lambda g, q: (g, 0)),
        pl.BlockSpec((_RB, 1), lambda g, q: (g, 0)),
    ],
    out_shape=[
        jax.ShapeDtypeStruct((4, _P, _Q), jnp.float32),
        jax.ShapeDtypeStruct((_P, 1), jnp.float32),
        jax.ShapeDtypeStruct((_P, 1), jnp.float32),
    ],
)


def _tc3_body(a0, a1, a2, a3, emb, dinv, out):
    qs = [jnp.sum(a[...], axis=0) for a in (a0, a1, a2, a3)]
    wide = jnp.concatenate(qs, axis=-1)
    out[...] = (1.0 / ((_NUM_LAYERS + 1) ** 2)) * (emb[...] + dinv[...] * wide)


_tc3 = pl.pallas_call(
    _tc3_body,
    grid=(_G,),
    in_specs=[
        pl.BlockSpec((_NUM_LAYERS, _RB, _Q),
                     functools.partial(lambda q, g: (0, q * _G + g, 0), q))
        for q in range(4)
    ]
    + [
        pl.BlockSpec((_RB, _DIM), lambda g: (g, 0)),
        pl.BlockSpec((_RB, 1), lambda g: (g, 0)),
    ],
    out_specs=pl.BlockSpec((_RB, _DIM), lambda g: (g, 0)),
    out_shape=jax.ShapeDtypeStruct((_P, _DIM), jnp.float32),
)


def kernel(edge_index, user_weight, item_weight):
    src = edge_index[0].astype(jnp.int32)
    dst = edge_index[1].astype(jnp.int32)
    pad = _EPAD - _N_EDGES
    src_p = jnp.concatenate([src, jnp.zeros((pad,), jnp.int32)])
    dst_p = jnp.concatenate([dst, jnp.full((pad,), _DUMP, jnp.int32)])
    offs = jnp.arange(4, dtype=jnp.int32)[:, None] * _P
    src4 = (src_p[None, :] + offs).reshape(4, 16, _NB, 128)
    dst_r = dst_p.reshape(16, _NB, 128)
    emb = jnp.concatenate(
        [user_weight, item_weight,
         jnp.zeros((_P - _N_NODES, _DIM), jnp.float32)], axis=0)

    degp = _deg_kernel(dst_r)
    z0, dinv, d2 = _tc1(degp, emb)
    d2r = d2[:, 0].reshape(16, _ROWS_W)
    acc, _ = _mega_kernel(z0.reshape(4 * _P, _Q), src4, dst_r, d2r)
    out = _tc3(acc, acc, acc, acc, emb, dinv)
    return out[:_NUM_USERS], out[_NUM_USERS:_N_NODES]
